# baseline (speedup 1.0000x reference)
import functools

import jax
import jax.numpy as jnp
from jax import lax
from jax.experimental import pallas as pl
from jax.experimental.pallas import tpu as pltpu
from jax.experimental.pallas import tpu_sc as plsc

_B = 16384
_D = 1000
_NSPL = 4
_BS = _B // _NSPL

_NC = 2
_NS = 16
_NW = _NC * _NS
_BPW = _BS // _NW
_FPW = _BPW * _D
_SR = 2 * _D
_CH = 8 * _SR
_NCHK = _FPW // _CH
_NACC = 8
_NCHUNK = _BPW // 16
_NIDX = _BPW // 128


_UR = 4


@functools.partial(
    pl.kernel,
    mesh=plsc.VectorSubcoreMesh(core_axis_name="c", subcore_axis_name="s"),
    out_type=jax.ShapeDtypeStruct((_NW, 16), jnp.float32),
    scratch_types=[
        pltpu.VMEM((4 * _CH,), jnp.float32),
        pltpu.VMEM((_SR,), jnp.float32),
        pltpu.VMEM((_BPW,), jnp.int32),
        pltpu.VMEM((_NIDX, 128), jnp.int32),
        pltpu.VMEM((_NIDX, 128), jnp.int32),
        pltpu.VMEM((_NIDX, 128), jnp.float32),
        pltpu.VMEM((_NIDX, 128), jnp.float32),
        pltpu.VMEM((16,), jnp.float32),
        pltpu.SemaphoreType.DMA,
        pltpu.SemaphoreType.DMA,
    ],
)
def _loss_kernel(xflat_hbm, tgt_hbm, w_hbm, w2_hbm, out_hbm,
                 xb_v, w2_v, tgt_v, idx_v, tdx_v, xs_v, ws_v, o_v,
                 gsem, ssem):
    wid = lax.axis_index("s") * _NC + lax.axis_index("c")
    base = wid * _FPW

    pltpu.sync_copy(tgt_hbm.at[pl.ds(wid * _BPW, _BPW)], tgt_v)
    for i in range(_NCHUNK):
        t16 = tgt_v[pl.ds(i * 16, 16)]
        rows = wid * _BPW + i * 16 + lax.broadcasted_iota(jnp.int32, (16,), 0)
        idx_v[i // 8, pl.ds((i % 8) * 16, 16)] = rows * _D + t16
        tdx_v[i // 8, pl.ds((i % 8) * 16, 16)] = t16
    gcopies = [
        pltpu.async_copy(xflat_hbm.at[idx_v.at[j]], xs_v.at[j], gsem)
        for j in range(_NIDX)
    ] + [
        pltpu.async_copy(w_hbm.at[tdx_v.at[j]], ws_v.at[j], gsem)
        for j in range(_NIDX)
    ]

    pltpu.sync_copy(w2_hbm, w2_v)
    for p in range(3):
        pltpu.async_copy(
            xflat_hbm.at[pl.ds(base + p * _CH, _CH)],
            xb_v.at[pl.ds(p * _CH, _CH)], ssem)

    def _chunk_body(k, accs):
        half = lax.rem(k, 4)

        @pl.when(k + 3 < _NCHK)
        def _start_next():
            pltpu.async_copy(
                xflat_hbm.at[pl.ds(base + (k + 3) * _CH, _CH)],
                xb_v.at[pl.ds(lax.rem(k + 3, 4) * _CH, _CH)], ssem)

        pltpu.make_async_copy(
            xflat_hbm.at[pl.ds(0, _CH)], xb_v.at[pl.ds(0, _CH)], ssem).wait()

        def _group_body(g, accs):
            accs = list(accs)
            off0 = half * _CH + g * (_UR * _SR)
            n = 0
            for c2 in range(_SR // 16):
                wv = w2_v[pl.ds(c2 * 16, 16)]
                for u in range(_UR):
                    xv = xb_v[pl.ds(off0 + u * _SR + c2 * 16, 16)]
                    accs[n % _NACC] = accs[n % _NACC] + wv * (xv * xv)
                    n += 1
            return tuple(accs)

        return lax.fori_loop(0, _CH // (_UR * _SR), _group_body, accs)

    accs = lax.fori_loop(
        0, _NCHK, _chunk_body,
        tuple(jnp.zeros((16,), jnp.float32) for _ in range(_NACC)))

    for cp in gcopies:
        cp.wait()
    acc = accs[0]
    for a in accs[1:]:
        acc = acc + a
    for i in range(_NCHUNK):
        x16 = xs_v[i // 8, pl.ds((i % 8) * 16, 16)]
        w16 = ws_v[i // 8, pl.ds((i % 8) * 16, 16)]
        acc = acc + w16 * (1.0 - 2.0 * x16)
    o_v[...] = acc
    pltpu.sync_copy(o_v, out_hbm.at[wid])


def kernel(inputs, targets, loss_weights):
    w2 = jnp.concatenate([loss_weights, loss_weights])
    total = jnp.float32(0.0)
    for s in range(_NSPL):
        xs = inputs[s * _BS:(s + 1) * _BS].reshape(_BS * _D)
        ts = targets[s * _BS:(s + 1) * _BS]
        total = total + jnp.sum(_loss_kernel(xs, ts, loss_weights, w2))
    return total / jnp.float32(_B * _D)


# speedup vs baseline: 1.2382x; 1.2382x over previous
import functools

import jax
import jax.numpy as jnp
from jax import lax
from jax.experimental import pallas as pl
from jax.experimental.pallas import tpu as pltpu
from jax.experimental.pallas import tpu_sc as plsc

_B = 16384
_D = 1000

_NC = 2
_NS = 16
_NW = _NC * _NS
_BPW = _B // _NW
_FPW = _BPW * _D
_SR = 2 * _D
_CH = 16 * _SR
_NCHK = _FPW // _CH
_NACC = 8
_NCHUNK = _BPW // 16
_NIDX = _BPW // 128


_UR = 4


@functools.partial(
    pl.kernel,
    mesh=plsc.VectorSubcoreMesh(core_axis_name="c", subcore_axis_name="s"),
    out_type=jax.ShapeDtypeStruct((_NW, 16), jnp.float32),
    scratch_types=[
        pltpu.VMEM((2 * _CH,), jnp.float32),
        pltpu.VMEM((_SR,), jnp.float32),
        pltpu.VMEM((_BPW,), jnp.int32),
        pltpu.VMEM((_NIDX, 128), jnp.int32),
        pltpu.VMEM((_NIDX, 128), jnp.int32),
        pltpu.VMEM((_NIDX, 128), jnp.float32),
        pltpu.VMEM((_NIDX, 128), jnp.float32),
        pltpu.VMEM((16,), jnp.float32),
        pltpu.SemaphoreType.DMA,
        pltpu.SemaphoreType.DMA,
    ],
)
def _loss_kernel(xflat_hbm, tgt_hbm, w_hbm, w2_hbm, out_hbm,
                 xb_v, w2_v, tgt_v, idx_v, tdx_v, xs_v, ws_v, o_v,
                 gsem, ssem):
    wid = lax.axis_index("s") * _NC + lax.axis_index("c")
    base = wid * _FPW

    pltpu.sync_copy(tgt_hbm.at[pl.ds(wid * _BPW, _BPW)], tgt_v)
    for i in range(_NCHUNK):
        t16 = tgt_v[pl.ds(i * 16, 16)]
        rows = wid * _BPW + i * 16 + lax.broadcasted_iota(jnp.int32, (16,), 0)
        idx_v[i // 8, pl.ds((i % 8) * 16, 16)] = rows * _D + t16
        tdx_v[i // 8, pl.ds((i % 8) * 16, 16)] = t16
    gcopies = [
        pltpu.async_copy(xflat_hbm.at[idx_v.at[j]], xs_v.at[j], gsem)
        for j in range(_NIDX)
    ] + [
        pltpu.async_copy(w_hbm.at[tdx_v.at[j]], ws_v.at[j], gsem)
        for j in range(_NIDX)
    ]

    pltpu.sync_copy(w2_hbm, w2_v)
    pltpu.async_copy(
        xflat_hbm.at[pl.ds(base, _CH)], xb_v.at[pl.ds(0, _CH)], ssem)

    def _chunk_body(k, accs):
        half = lax.rem(k, 2)

        @pl.when(k + 1 < _NCHK)
        def _start_next():
            pltpu.async_copy(
                xflat_hbm.at[pl.ds(base + (k + 1) * _CH, _CH)],
                xb_v.at[pl.ds(lax.rem(k + 1, 2) * _CH, _CH)], ssem)

        pltpu.make_async_copy(
            xflat_hbm.at[pl.ds(0, _CH)], xb_v.at[pl.ds(0, _CH)], ssem).wait()

        def _group_body(g, accs):
            accs = list(accs)
            off0 = half * _CH + g * (_UR * _SR)
            n = 0
            for c2 in range(_SR // 16):
                wv = w2_v[pl.ds(c2 * 16, 16)]
                for u in range(_UR):
                    xv = xb_v[pl.ds(off0 + u * _SR + c2 * 16, 16)]
                    accs[n % _NACC] = accs[n % _NACC] + wv * (xv * xv)
                    n += 1
            return tuple(accs)

        return lax.fori_loop(0, _CH // (_UR * _SR), _group_body, accs)

    accs = lax.fori_loop(
        0, _NCHK, _chunk_body,
        tuple(jnp.zeros((16,), jnp.float32) for _ in range(_NACC)))

    for cp in gcopies:
        cp.wait()
    acc = accs[0]
    for a in accs[1:]:
        acc = acc + a
    for i in range(_NCHUNK):
        x16 = xs_v[i // 8, pl.ds((i % 8) * 16, 16)]
        w16 = ws_v[i // 8, pl.ds((i % 8) * 16, 16)]
        acc = acc + w16 * (1.0 - 2.0 * x16)
    o_v[...] = acc
    pltpu.sync_copy(o_v, out_hbm.at[wid])


def kernel(inputs, targets, loss_weights):
    xflat = inputs.reshape(_B * _D)
    w2 = jnp.concatenate([loss_weights, loss_weights])
    parts = _loss_kernel(xflat, targets, loss_weights, w2)
    return jnp.sum(parts) / jnp.float32(_B * _D)
